# baseline (speedup 1.0000x reference)
import functools

import jax
import jax.numpy as jnp
from jax import lax
from jax.experimental import pallas as pl
from jax.experimental.pallas import tpu as pltpu
from jax.experimental.pallas import tpu_sc as plsc

_LANES = 16
_IDX_CHUNK = 128


def _make_sc_gather(batch, n_actions):
    info = plsc.get_sparse_core_info()
    num_workers = info.num_cores * info.num_subcores
    b_per_w = batch // num_workers
    n_chunks = b_per_w // _IDX_CHUNK
    mesh = plsc.VectorSubcoreMesh(core_axis_name="c", subcore_axis_name="s")

    @functools.partial(
        pl.kernel,
        mesh=mesh,
        out_type=jax.ShapeDtypeStruct((batch,), jnp.float32),
        scratch_types=[
            pltpu.VMEM((b_per_w,), jnp.int32),
            pltpu.VMEM((b_per_w,), jnp.int32),
            pltpu.VMEM((b_per_w,), jnp.int32),
            pltpu.VMEM((b_per_w,), jnp.float32),
            pltpu.SemaphoreType.DMA,
        ] + [pltpu.SemaphoreType.DMA] * (2 * n_chunks),
    )
    def sc_gather(state_hbm, action_hbm, table_hbm, out_hbm,
                  s_v, a_v, idx_v, vals_v, w_sem, *sems):
        in_sems, g_sems = sems[:n_chunks], sems[n_chunks:]
        wid = lax.axis_index("s") * info.num_cores + lax.axis_index("c")
        base = wid * b_per_w

        in_copies = []
        for j in range(n_chunks):
            chunk = pl.ds(j * _IDX_CHUNK, _IDX_CHUNK)
            src = pl.ds(base + j * _IDX_CHUNK, _IDX_CHUNK)
            pair = [
                pltpu.make_async_copy(state_hbm.at[src], s_v.at[chunk], in_sems[j]),
                pltpu.make_async_copy(action_hbm.at[src], a_v.at[chunk], in_sems[j]),
            ]
            for c in pair:
                c.start()
            in_copies.append(pair)

        log2_actions = n_actions.bit_length() - 1
        assert n_actions == 1 << log2_actions

        gathers = []
        for j in range(n_chunks):
            for c in in_copies[j]:
                c.wait()
            for i in range(_IDX_CHUNK // _LANES):
                sl = pl.ds(j * _IDX_CHUNK + i * _LANES, _LANES)
                idx_v[sl] = (s_v[sl] << log2_actions) | a_v[sl]
            chunk = pl.ds(j * _IDX_CHUNK, _IDX_CHUNK)
            g = pltpu.make_async_copy(
                table_hbm.at[idx_v.at[chunk]], vals_v.at[chunk], g_sems[j])
            g.start()
            gathers.append(g)

        writebacks = []
        for j in range(n_chunks):
            gathers[j].wait()
            chunk = pl.ds(j * _IDX_CHUNK, _IDX_CHUNK)
            w = pltpu.make_async_copy(
                vals_v.at[chunk],
                out_hbm.at[pl.ds(base + j * _IDX_CHUNK, _IDX_CHUNK)], w_sem)
            w.start()
            writebacks.append(w)
        for w in writebacks:
            w.wait()

    return sc_gather


def kernel(state, action, table):
    batch = state.shape[0]
    n_actions = table.shape[1]
    flat_table = table.reshape(-1)
    fn = _make_sc_gather(batch, n_actions)
    return fn(state.astype(jnp.int32), action.astype(jnp.int32), flat_table)


# speedup vs baseline: 1.0127x; 1.0127x over previous
import functools

import jax
import jax.numpy as jnp
from jax import lax
from jax.experimental import pallas as pl
from jax.experimental.pallas import tpu as pltpu
from jax.experimental.pallas import tpu_sc as plsc

_LANES = 16
_IDX_CHUNK = 128


def _make_sc_gather(batch, n_actions):
    info = plsc.get_sparse_core_info()
    num_workers = info.num_cores * info.num_subcores
    b_per_w = batch // num_workers
    n_chunks = b_per_w // _IDX_CHUNK
    mesh = plsc.VectorSubcoreMesh(core_axis_name="c", subcore_axis_name="s")

    @functools.partial(
        pl.kernel,
        mesh=mesh,
        out_type=jax.ShapeDtypeStruct((batch,), jnp.float32),
        scratch_types=[
            pltpu.VMEM((b_per_w,), jnp.int32),
            pltpu.VMEM((b_per_w,), jnp.int32),
            pltpu.VMEM((b_per_w,), jnp.int32),
            pltpu.VMEM((b_per_w,), jnp.float32),
            pltpu.SemaphoreType.DMA,
            pltpu.SemaphoreType.DMA,
        ] + [pltpu.SemaphoreType.DMA] * n_chunks,
    )
    def sc_gather(state_hbm, action_hbm, table_hbm, out_hbm,
                  s_v, a_v, idx_v, vals_v, in_sem, w_sem, *g_sems):
        wid = lax.axis_index("s") * info.num_cores + lax.axis_index("c")
        base = wid * b_per_w

        in_copies = [
            pltpu.make_async_copy(
                state_hbm.at[pl.ds(base, b_per_w)], s_v, in_sem),
            pltpu.make_async_copy(
                action_hbm.at[pl.ds(base, b_per_w)], a_v, in_sem),
        ]
        for c in in_copies:
            c.start()
        for c in in_copies:
            c.wait()

        log2_actions = n_actions.bit_length() - 1
        assert n_actions == 1 << log2_actions

        gathers = []
        for j in range(n_chunks):
            for i in range(_IDX_CHUNK // _LANES):
                sl = pl.ds(j * _IDX_CHUNK + i * _LANES, _LANES)
                idx_v[sl] = (s_v[sl] << log2_actions) | a_v[sl]
            chunk = pl.ds(j * _IDX_CHUNK, _IDX_CHUNK)
            g = pltpu.make_async_copy(
                table_hbm.at[idx_v.at[chunk]], vals_v.at[chunk], g_sems[j])
            g.start()
            gathers.append(g)

        writebacks = []
        for j in range(n_chunks):
            gathers[j].wait()
            chunk = pl.ds(j * _IDX_CHUNK, _IDX_CHUNK)
            w = pltpu.make_async_copy(
                vals_v.at[chunk],
                out_hbm.at[pl.ds(base + j * _IDX_CHUNK, _IDX_CHUNK)], w_sem)
            w.start()
            writebacks.append(w)
        for w in writebacks:
            w.wait()

    return sc_gather


def kernel(state, action, table):
    batch = state.shape[0]
    n_actions = table.shape[1]
    flat_table = table.reshape(-1)
    fn = _make_sc_gather(batch, n_actions)
    return fn(state.astype(jnp.int32), action.astype(jnp.int32), flat_table)


# speedup vs baseline: 1.0145x; 1.0018x over previous
import functools

import jax
import jax.numpy as jnp
from jax import lax
from jax.experimental import pallas as pl
from jax.experimental.pallas import tpu as pltpu
from jax.experimental.pallas import tpu_sc as plsc

_LANES = 16
_IDX_CHUNK = 128


def _make_sc_gather(batch, n_actions):
    info = plsc.get_sparse_core_info()
    num_workers = info.num_cores * info.num_subcores
    b_per_w = batch // num_workers
    n_chunks = b_per_w // _IDX_CHUNK
    mesh = plsc.VectorSubcoreMesh(core_axis_name="c", subcore_axis_name="s")

    @functools.partial(
        pl.kernel,
        mesh=mesh,
        out_type=jax.ShapeDtypeStruct((batch,), jnp.float32),
        scratch_types=[
            pltpu.VMEM((b_per_w,), jnp.int32),
            pltpu.VMEM((b_per_w,), jnp.int32),
            pltpu.VMEM((b_per_w,), jnp.int32),
            pltpu.VMEM((b_per_w,), jnp.float32),
            pltpu.SemaphoreType.DMA,
            pltpu.SemaphoreType.DMA,
        ] + [pltpu.SemaphoreType.DMA] * n_chunks,
    )
    def sc_gather(state_hbm, action_hbm, table_hbm, out_hbm,
                  s_v, a_v, idx_v, vals_v, in_sem, w_sem, *g_sems):
        wid = lax.axis_index("s") * info.num_cores + lax.axis_index("c")
        base = wid * b_per_w

        in_copies = [
            pltpu.make_async_copy(
                state_hbm.at[pl.ds(base, b_per_w)], s_v, in_sem),
            pltpu.make_async_copy(
                action_hbm.at[pl.ds(base, b_per_w)], a_v, in_sem),
        ]
        with jax.named_scope("ph_in"):
            for c in in_copies:
                c.start()
            for c in in_copies:
                c.wait()

        log2_actions = n_actions.bit_length() - 1
        assert n_actions == 1 << log2_actions

        with jax.named_scope("ph_idx_fire"):
            gathers = []
            for j in range(n_chunks):
                for i in range(_IDX_CHUNK // _LANES):
                    sl = pl.ds(j * _IDX_CHUNK + i * _LANES, _LANES)
                    idx_v[sl] = (s_v[sl] << log2_actions) | a_v[sl]
                chunk = pl.ds(j * _IDX_CHUNK, _IDX_CHUNK)
                g = pltpu.make_async_copy(
                    table_hbm.at[idx_v.at[chunk]], vals_v.at[chunk], g_sems[j])
                g.start()
                gathers.append(g)

        with jax.named_scope("ph_drain_wb"):
            writebacks = []
            for j in range(n_chunks):
                gathers[j].wait()
                chunk = pl.ds(j * _IDX_CHUNK, _IDX_CHUNK)
                w = pltpu.make_async_copy(
                    vals_v.at[chunk],
                    out_hbm.at[pl.ds(base + j * _IDX_CHUNK, _IDX_CHUNK)], w_sem)
                w.start()
                writebacks.append(w)
            for w in writebacks:
                w.wait()

    return sc_gather


def kernel(state, action, table):
    batch = state.shape[0]
    n_actions = table.shape[1]
    flat_table = table.reshape(-1)
    fn = _make_sc_gather(batch, n_actions)
    return fn(state.astype(jnp.int32), action.astype(jnp.int32), flat_table)
